# baseline (speedup 1.0000x reference)
import functools

import jax
import jax.numpy as jnp
from jax import lax
from jax.experimental import pallas as pl
from jax.experimental.pallas import tpu as pltpu
from jax.experimental.pallas import tpu_sc as plsc

_NUM_CORES = 2
_NUM_SUBCORES = 16
_NW = _NUM_CORES * _NUM_SUBCORES
_HPAD = 56


def kernel(indices, weight):
    b, h = indices.shape
    vocab, d = weight.shape
    batches_per_w = b // _NW
    ntile, tail = divmod(h, 8)
    idx_pad = jnp.pad(indices.astype(jnp.int32), ((0, 0), (0, _HPAD - h)))
    idx_flat = idx_pad.reshape(b * _HPAD)
    per_w_idx = batches_per_w * _HPAD

    mesh = plsc.VectorSubcoreMesh(core_axis_name="c", subcore_axis_name="s")

    @functools.partial(
        pl.kernel,
        mesh=mesh,
        out_type=jax.ShapeDtypeStruct((b, h, d), jnp.float32),
        scratch_types=[
            pltpu.VMEM((per_w_idx,), jnp.int32),
            pltpu.VMEM((8, d), jnp.float32),
            pltpu.SemaphoreType.DMA,
        ],
    )
    def gather_rows(idx_hbm, table_hbm, out_hbm, idx_v, rows_v, sem_g):
        wid = lax.axis_index("s") * _NUM_CORES + lax.axis_index("c")
        base = wid * batches_per_w
        pltpu.sync_copy(idx_hbm.at[pl.ds(base * _HPAD, per_w_idx)], idx_v)

        def body(j, carry):
            bb = base + j
            for r in range(ntile):
                pltpu.async_copy(
                    table_hbm.at[idx_v.at[pl.ds(j * _HPAD + r * 8, 8)]],
                    rows_v,
                    sem_g,
                ).wait()
                pltpu.sync_copy(rows_v, out_hbm.at[bb, pl.ds(r * 8, 8)])
            if tail:
                pltpu.async_copy(
                    table_hbm.at[idx_v.at[pl.ds(j * _HPAD + ntile * 8, 8)]],
                    rows_v,
                    sem_g,
                ).wait()
                pltpu.sync_copy(
                    rows_v.at[pl.ds(0, tail)],
                    out_hbm.at[bb, pl.ds(ntile * 8, tail)],
                )
            return carry

        lax.fori_loop(0, batches_per_w, body, 0)

    return gather_rows(idx_flat, weight)
